# baseline (speedup 1.0000x reference)
import functools

import numpy as np

import jax
import jax.numpy as jnp
from jax import lax
from jax.experimental import pallas as pl
from jax.experimental.pallas import tpu as pltpu
from jax.experimental.pallas import tpu_sc as plsc

B, N, D = 4, 4096, 1024
ROWS = B * N
NW = 32
RPW = ROWS // NW
C = 32
NCH = RPW // C

K = 3

_mesh = plsc.VectorSubcoreMesh(core_axis_name="c", subcore_axis_name="s")


@functools.partial(
    pl.kernel,
    mesh=_mesh,
    out_type=jax.ShapeDtypeStruct((NW, NCH, C, D), jnp.float32),
    scratch_types=[
        pltpu.VMEM((NCH, C), jnp.int32),
        pltpu.VMEM((C, D), jnp.float32),
        pltpu.VMEM((C, D), jnp.float32),
        pltpu.VMEM((C, D), jnp.float32),
        pltpu.SemaphoreType.DMA,
        pltpu.SemaphoreType.DMA,
        pltpu.SemaphoreType.DMA,
        pltpu.SemaphoreType.DMA,
        pltpu.SemaphoreType.DMA,
        pltpu.SemaphoreType.DMA,
    ],
)
def _shuffle_sc(x_hbm, idx_hbm, out_hbm, idx_v, b0, b1, b2,
                gs0, gs1, gs2, ws0, ws1, ws2):
    wid = lax.axis_index("s") * 2 + lax.axis_index("c")
    pltpu.sync_copy(idx_hbm.at[wid], idx_v)
    bufs = (b0, b1, b2)
    gsems = (gs0, gs1, gs2)
    wsems = (ws0, ws1, ws2)
    g = [None] * NCH
    w = [None] * NCH
    for j in range(K):
        g[j] = pltpu.async_copy(x_hbm.at[idx_v.at[j]], bufs[j], gsems[j])
    for j in range(NCH):
        s = j % K
        if j >= 1 and j - 1 + K < NCH:
            p = j - 1
            w[p].wait()
            g[p + K] = pltpu.async_copy(
                x_hbm.at[idx_v.at[p + K]], bufs[p % K], gsems[p % K])
        g[j].wait()
        w[j] = pltpu.async_copy(bufs[s], out_hbm.at[wid, j], wsems[s])
    for j in range(NCH - K, NCH):
        if w[j] is not None:
            w[j].wait()


_IDX_CACHE = {}


def _flat_indices(n):
    if n not in _IDX_CACHE:
        with jax.ensure_compile_time_eval():
            perm = np.asarray(
                jax.random.permutation(jax.random.key(42), n)).astype(np.int32)
        base = (np.arange(B, dtype=np.int32) * n)[:, None]
        idx_sh = (base + perm[None, :]).reshape(NW, NCH, C)
        idx_id = (base + np.arange(n, dtype=np.int32)[None, :]).reshape(
            NW, NCH, C)
        _IDX_CACHE[n] = (idx_sh, idx_id)
    return _IDX_CACHE[n]


G = 8


def _tc_body(idx_ref, *refs):
    ins, out = refs[:G], refs[G]
    for k in range(G):
        out[k:k + 1, :] = ins[k][0]


def _shuffle_tc(x_flat, idx_flat, nrows):
    x3 = x_flat.reshape(nrows, 1, D)
    grid_spec = pltpu.PrefetchScalarGridSpec(
        num_scalar_prefetch=1,
        grid=(nrows // G,),
        in_specs=[
            pl.BlockSpec((1, 1, D),
                         lambda i, idx_ref, k=k: (idx_ref[i * G + k], 0, 0))
            for k in range(G)
        ],
        out_specs=pl.BlockSpec((G, D), lambda i, idx_ref: (i, 0)),
    )
    return pl.pallas_call(
        _tc_body,
        grid_spec=grid_spec,
        out_shape=jax.ShapeDtypeStruct((nrows, D), jnp.float32),
    )(idx_flat, *([x3] * G))


def kernel(x, sample):
    b, n, d = x.shape
    idx_sh, idx_id = _flat_indices(n)
    idx = jnp.where(sample != 0, jnp.asarray(idx_sh), jnp.asarray(idx_id))
    out = _shuffle_tc(x.reshape(b * n, d), idx.reshape(-1), b * n)
    return out.reshape(b, n, d)


# speedup vs baseline: 17.9362x; 17.9362x over previous
import functools

import numpy as np

import jax
import jax.numpy as jnp
from jax import lax
from jax.experimental import pallas as pl
from jax.experimental.pallas import tpu as pltpu
from jax.experimental.pallas import tpu_sc as plsc

B, N, D = 4, 4096, 1024
ROWS = B * N
NW = 32
RPW = ROWS // NW
C = 32
NCH = RPW // C

K = 3

_mesh = plsc.VectorSubcoreMesh(core_axis_name="c", subcore_axis_name="s")


@functools.partial(
    pl.kernel,
    mesh=_mesh,
    out_type=jax.ShapeDtypeStruct((NW, NCH, C, D), jnp.float32),
    scratch_types=[
        pltpu.VMEM((2, NCH, C), jnp.int32),
        pltpu.VMEM((16,), jnp.int32),
        pltpu.VMEM((NCH, C), jnp.int32),
        pltpu.VMEM((C, D), jnp.float32),
        pltpu.VMEM((C, D), jnp.float32),
        pltpu.VMEM((C, D), jnp.float32),
        pltpu.SemaphoreType.DMA,
        pltpu.SemaphoreType.DMA,
        pltpu.SemaphoreType.DMA,
        pltpu.SemaphoreType.DMA,
        pltpu.SemaphoreType.DMA,
        pltpu.SemaphoreType.DMA,
    ],
)
def _shuffle_sc(x_hbm, idxb_hbm, s_hbm, out_hbm, idxb_v, s_v, idx_v,
                b0, b1, b2, gs0, gs1, gs2, ws0, ws1, ws2):
    wid = lax.axis_index("s") * 2 + lax.axis_index("c")
    pltpu.sync_copy(idxb_hbm.at[wid], idxb_v)
    pltpu.sync_copy(s_hbm, s_v)
    s = s_v[...]
    for r in range(NCH):
        for o in range(C // 16):
            sl = pl.ds(o * 16, 16)
            idx_v[r, sl] = idxb_v[0, r, sl] + s * idxb_v[1, r, sl]
    bufs = (b0, b1, b2)
    gsems = (gs0, gs1, gs2)
    wsems = (ws0, ws1, ws2)
    g = [None] * NCH
    w = [None] * NCH
    for j in range(K):
        g[j] = pltpu.async_copy(x_hbm.at[idx_v.at[j]], bufs[j], gsems[j])
    for j in range(NCH):
        s = j % K
        if j >= 1 and j - 1 + K < NCH:
            p = j - 1
            w[p].wait()
            g[p + K] = pltpu.async_copy(
                x_hbm.at[idx_v.at[p + K]], bufs[p % K], gsems[p % K])
        g[j].wait()
        w[j] = pltpu.async_copy(bufs[s], out_hbm.at[wid, j], wsems[s])
    for j in range(NCH - K, NCH):
        if w[j] is not None:
            w[j].wait()


_IDX_CACHE = {}


def _flat_indices(n):
    if n not in _IDX_CACHE:
        with jax.ensure_compile_time_eval():
            perm = np.asarray(
                jax.random.permutation(jax.random.key(42), n)).astype(np.int32)
        base = (np.arange(B, dtype=np.int32) * n)[:, None]
        idx_sh = (base + perm[None, :]).reshape(NW, 1, NCH, C)
        idx_id = (base + np.arange(n, dtype=np.int32)[None, :]).reshape(
            NW, 1, NCH, C)
        _IDX_CACHE[n] = np.concatenate([idx_id, idx_sh - idx_id], axis=1)
    return _IDX_CACHE[n]


def kernel(x, sample):
    b, n, d = x.shape
    idx_both = jnp.asarray(_flat_indices(n))
    gate = jnp.broadcast_to((sample != 0).astype(jnp.int32), (16,))
    out = _shuffle_sc(x.reshape(b * n, d), idx_both, gate)
    return out.reshape(b, n, d)


# speedup vs baseline: 18.4496x; 1.0286x over previous
import functools

import numpy as np

import jax
import jax.numpy as jnp
from jax import lax
from jax.experimental import pallas as pl
from jax.experimental.pallas import tpu as pltpu
from jax.experimental.pallas import tpu_sc as plsc

B, N, D = 4, 4096, 1024
ROWS = B * N
NW = 32
RPW = ROWS // NW
C = 16
NCH = RPW // C

K = 6

_mesh = plsc.VectorSubcoreMesh(core_axis_name="c", subcore_axis_name="s")


@functools.partial(
    pl.kernel,
    mesh=_mesh,
    out_type=jax.ShapeDtypeStruct((NW, NCH, C, D), jnp.float32),
    scratch_types=(
        [pltpu.VMEM((NCH, C), jnp.int32)]
        + [pltpu.VMEM((C, D), jnp.float32)] * K
        + [pltpu.SemaphoreType.DMA] * (2 * K)
    ),
)
def _shuffle_sc(x_hbm, idx_hbm, out_hbm, idx_v, *scratch):
    wid = lax.axis_index("s") * 2 + lax.axis_index("c")
    pltpu.sync_copy(idx_hbm.at[wid], idx_v)
    bufs = scratch[:K]
    gsems = scratch[K:2 * K]
    wsems = scratch[2 * K:3 * K]
    g = [None] * NCH
    w = [None] * NCH
    for j in range(K):
        g[j] = pltpu.async_copy(x_hbm.at[idx_v.at[j]], bufs[j], gsems[j])
    for j in range(NCH):
        s = j % K
        if j >= 1 and j - 1 + K < NCH:
            p = j - 1
            w[p].wait()
            g[p + K] = pltpu.async_copy(
                x_hbm.at[idx_v.at[p + K]], bufs[p % K], gsems[p % K])
        g[j].wait()
        w[j] = pltpu.async_copy(bufs[s], out_hbm.at[wid, j], wsems[s])
    for j in range(NCH - K, NCH):
        if w[j] is not None:
            w[j].wait()


_IDX_CACHE = {}


def _flat_indices(n):
    if n not in _IDX_CACHE:
        with jax.ensure_compile_time_eval():
            perm = np.asarray(
                jax.random.permutation(jax.random.key(42), n)).astype(np.int32)
        base = (np.arange(B, dtype=np.int32) * n)[:, None]
        idx_sh = (base + perm[None, :]).reshape(NW, NCH, C)
        idx_id = (base + np.arange(n, dtype=np.int32)[None, :]).reshape(
            NW, NCH, C)
        _IDX_CACHE[n] = (idx_sh, idx_id)
    return _IDX_CACHE[n]


def kernel(x, sample):
    b, n, d = x.shape
    idx_sh, idx_id = _flat_indices(n)
    idx = jnp.where(sample != 0, jnp.asarray(idx_sh), jnp.asarray(idx_id))
    out = _shuffle_sc(x.reshape(b * n, d), idx)
    return out.reshape(b, n, d)
